# initial kernel scaffold (unmeasured)
import jax
import jax.numpy as jnp
from jax import lax
from jax.experimental import pallas as pl
from jax.experimental.pallas import tpu as pltpu

T, D, V = 2048, 4096, 16384
RB = T // 4
CB = V // 2


def kernel(x, W):
    xi = lax.axis_index("x")
    yi = lax.axis_index("y")
    zi = lax.axis_index("z")
    r = 2 * xi + zi
    x_rows = lax.dynamic_slice_in_dim(x, r * RB, RB, axis=0)
    logits = jnp.dot(x_rows, W, preferred_element_type=jnp.float32)

    def body(logits_ref, out_ref, e_ref, st_send, st_recv,
             send_sems, recv_sems, copy_sem):
        xi = lax.axis_index("x")
        yi = lax.axis_index("y")
        zi = lax.axis_index("z")
        row0 = (2 * xi + zi) * RB
        col0 = yi * CB
        xrow0 = xi * (2 * RB)

        barrier = pltpu.get_barrier_semaphore()
        for nbr in [(1 - xi, yi, zi), (xi, 1 - yi, zi), (xi, yi, 1 - zi)]:
            pl.semaphore_signal(barrier, inc=1, device_id=nbr,
                                device_id_type=pl.DeviceIdType.MESH)
        pl.semaphore_wait(barrier, 3)

        l = logits_ref[:, :]
        m = jnp.max(l, axis=1, keepdims=True)
        e = jnp.exp(l - m)
        e_ref[:, :] = e
        s = jnp.sum(e, axis=1, keepdims=True)
        st_send[:, 0:128] = jnp.broadcast_to(m, (RB, 128))
        st_send[:, 128:256] = jnp.broadcast_to(s, (RB, 128))

        stats_rdma = pltpu.make_async_remote_copy(
            src_ref=st_send, dst_ref=st_recv,
            send_sem=send_sems.at[0], recv_sem=recv_sems.at[0],
            device_id=(xi, 1 - yi, zi), device_id_type=pl.DeviceIdType.MESH)
        stats_rdma.start()
        stats_rdma.wait()

        m_o = st_recv[:, 0:1]
        s_o = st_recv[:, 128:129]
        m_g = jnp.maximum(m, m_o)
        s_g = s * jnp.exp(m - m_g) + s_o * jnp.exp(m_o - m_g)
        e_ref[:, :] = e_ref[:, :] * (jnp.exp(m - m_g) / s_g)

        local = pltpu.make_async_copy(
            e_ref, out_ref.at[pl.ds(row0, RB), pl.ds(col0, CB)], copy_sem)
        local.start()
        local.wait()

        z_rdma = pltpu.make_async_remote_copy(
            src_ref=e_ref,
            dst_ref=out_ref.at[pl.ds(row0, RB), pl.ds(col0, CB)],
            send_sem=send_sems.at[1], recv_sem=recv_sems.at[1],
            device_id=(xi, yi, 1 - zi), device_id_type=pl.DeviceIdType.MESH)
        z_rdma.start()
        z_rdma.wait()

        y_rdma = pltpu.make_async_remote_copy(
            src_ref=out_ref.at[pl.ds(xrow0, 2 * RB), pl.ds(col0, CB)],
            dst_ref=out_ref.at[pl.ds(xrow0, 2 * RB), pl.ds(col0, CB)],
            send_sem=send_sems.at[2], recv_sem=recv_sems.at[2],
            device_id=(xi, 1 - yi, zi), device_id_type=pl.DeviceIdType.MESH)
        y_rdma.start()
        y_rdma.wait()

        x_rdma = pltpu.make_async_remote_copy(
            src_ref=out_ref.at[pl.ds(xrow0, 2 * RB)],
            dst_ref=out_ref.at[pl.ds(xrow0, 2 * RB)],
            send_sem=send_sems.at[3], recv_sem=recv_sems.at[3],
            device_id=(1 - xi, yi, zi), device_id_type=pl.DeviceIdType.MESH)
        x_rdma.start()
        x_rdma.wait()

    return pl.pallas_call(
        body,
        out_shape=jax.ShapeDtypeStruct((T, V), jnp.float32),
        in_specs=[pl.BlockSpec(memory_space=pltpu.VMEM)],
        out_specs=pl.BlockSpec(memory_space=pltpu.ANY),
        scratch_shapes=[
            pltpu.VMEM((RB, CB), jnp.float32),
            pltpu.VMEM((RB, 256), jnp.float32),
            pltpu.VMEM((RB, 256), jnp.float32),
            pltpu.SemaphoreType.DMA((4,)),
            pltpu.SemaphoreType.DMA((4,)),
            pltpu.SemaphoreType.DMA,
        ],
        compiler_params=pltpu.CompilerParams(collective_id=0),
    )(logits)


# baseline (device time: 1461370 ns/iter reference)
import jax
import jax.numpy as jnp
from jax import lax
from jax.experimental import pallas as pl
from jax.experimental.pallas import tpu as pltpu

T, D, V = 2048, 4096, 16384
RB = T // 4
CB = V // 2


def kernel(x, W):
    xi = lax.axis_index("x")
    yi = lax.axis_index("y")
    zi = lax.axis_index("z")
    r = 2 * xi + zi
    x_rows = lax.dynamic_slice_in_dim(x, r * RB, RB, axis=0)
    logits = jnp.dot(x_rows, W, preferred_element_type=jnp.float32)

    def body(logits_ref, out_ref, e_ref, st_send, st_recv,
             send_sems, recv_sems, copy_sem):
        xi = lax.axis_index("x")
        yi = lax.axis_index("y")
        zi = lax.axis_index("z")
        row0 = (2 * xi + zi) * RB
        col0 = yi * CB
        xrow0 = xi * (2 * RB)

        barrier = pltpu.get_barrier_semaphore()
        for nbr in [(1 - xi, yi, zi), (xi, 1 - yi, zi), (xi, yi, 1 - zi)]:
            pl.semaphore_signal(barrier, inc=1, device_id=nbr,
                                device_id_type=pl.DeviceIdType.MESH)
        pl.semaphore_wait(barrier, 3)

        l = logits_ref[:, :]
        m = jnp.max(l, axis=1, keepdims=True)
        e = jnp.exp(l - m)
        e_ref[:, :] = e
        s = jnp.sum(e, axis=1, keepdims=True)
        st_send[:, 0:128] = jnp.broadcast_to(m, (RB, 128))
        st_send[:, 128:256] = jnp.broadcast_to(s, (RB, 128))

        stats_rdma = pltpu.make_async_remote_copy(
            src_ref=st_send, dst_ref=st_recv,
            send_sem=send_sems.at[0], recv_sem=recv_sems.at[0],
            device_id=(xi, 1 - yi, zi), device_id_type=pl.DeviceIdType.MESH)
        stats_rdma.start()
        stats_rdma.wait()

        m_o = st_recv[:, 0:1]
        s_o = st_recv[:, 128:129]
        m_g = jnp.maximum(m, m_o)
        s_g = s * jnp.exp(m - m_g) + s_o * jnp.exp(m_o - m_g)
        e_ref[:, :] = e_ref[:, :] * (jnp.exp(m - m_g) / s_g)

        local = pltpu.make_async_copy(
            e_ref, out_ref.at[pl.ds(row0, RB), pl.ds(col0, CB)], copy_sem)
        local.start()
        local.wait()

        z_rdma = pltpu.make_async_remote_copy(
            src_ref=e_ref,
            dst_ref=out_ref.at[pl.ds(row0, RB), pl.ds(col0, CB)],
            send_sem=send_sems.at[1], recv_sem=recv_sems.at[1],
            device_id=(xi, yi, 1 - zi), device_id_type=pl.DeviceIdType.MESH)
        z_rdma.start()
        z_rdma.wait()

        y_rdma = pltpu.make_async_remote_copy(
            src_ref=out_ref.at[pl.ds(xrow0, 2 * RB), pl.ds(col0, CB)],
            dst_ref=out_ref.at[pl.ds(xrow0, 2 * RB), pl.ds(col0, CB)],
            send_sem=send_sems.at[2], recv_sem=recv_sems.at[2],
            device_id=(xi, 1 - yi, zi), device_id_type=pl.DeviceIdType.MESH)
        y_rdma.start()
        y_rdma.wait()

        x_rdma = pltpu.make_async_remote_copy(
            src_ref=out_ref.at[pl.ds(xrow0, 2 * RB)],
            dst_ref=out_ref.at[pl.ds(xrow0, 2 * RB)],
            send_sem=send_sems.at[3], recv_sem=recv_sems.at[3],
            device_id=(1 - xi, yi, zi), device_id_type=pl.DeviceIdType.MESH)
        x_rdma.start()
        x_rdma.wait()

    return pl.pallas_call(
        body,
        out_shape=jax.ShapeDtypeStruct((T, V), jnp.float32),
        in_specs=[pl.BlockSpec(memory_space=pltpu.MemorySpace.VMEM)],
        out_specs=pl.BlockSpec(memory_space=pl.ANY),
        scratch_shapes=[
            pltpu.VMEM((RB, CB), jnp.float32),
            pltpu.VMEM((RB, 256), jnp.float32),
            pltpu.VMEM((RB, 256), jnp.float32),
            pltpu.SemaphoreType.DMA((4,)),
            pltpu.SemaphoreType.DMA((4,)),
            pltpu.SemaphoreType.DMA,
        ],
        compiler_params=pltpu.CompilerParams(collective_id=0),
    )(logits)


# device time: 914428 ns/iter; 1.5981x vs baseline; 1.5981x over previous
import jax
import jax.numpy as jnp
from jax import lax
from jax.experimental import pallas as pl
from jax.experimental.pallas import tpu as pltpu

T, D, V = 2048, 4096, 16384
RB = T // 4
CB = V // 2

S_STATS, S_Z, S_YA, S_YB, S_XA, S_XB, S_XCA, S_XCB = range(8)


def kernel(x, W):
    xi = lax.axis_index("x")
    yi = lax.axis_index("y")
    zi = lax.axis_index("z")
    r = 2 * xi + zi
    x_rows = lax.dynamic_slice_in_dim(x, r * RB, RB, axis=0)
    logits = jnp.dot(x_rows, W, preferred_element_type=jnp.float32)

    def body(logits_ref, out_ref, e_ref, st_send, st_recv,
             send_sems, recv_sems, copy_sem):
        xi = lax.axis_index("x")
        yi = lax.axis_index("y")
        zi = lax.axis_index("z")
        row0 = (2 * xi + zi) * RB
        rowz = (2 * xi + (1 - zi)) * RB
        col0 = yi * CB
        colo = (1 - yi) * CB
        z_peer = (xi, yi, 1 - zi)
        y_peer = (xi, 1 - yi, zi)
        x_peer = (1 - xi, yi, zi)

        def rdma(src, dst, sem_idx, peer):
            return pltpu.make_async_remote_copy(
                src_ref=src, dst_ref=dst,
                send_sem=send_sems.at[sem_idx], recv_sem=recv_sems.at[sem_idx],
                device_id=peer, device_id_type=pl.DeviceIdType.MESH)

        barrier = pltpu.get_barrier_semaphore()
        for nbr in [x_peer, y_peer, z_peer]:
            pl.semaphore_signal(barrier, inc=1, device_id=nbr,
                                device_id_type=pl.DeviceIdType.MESH)
        pl.semaphore_wait(barrier, 3)

        l = logits_ref[:, :]
        m = jnp.max(l, axis=1, keepdims=True)
        e = jnp.exp(l - m)
        e_ref[:, :] = e
        s = jnp.sum(e, axis=1, keepdims=True)
        st_send[:, 0:128] = jnp.broadcast_to(m, (RB, 128))
        st_send[:, 128:256] = jnp.broadcast_to(s, (RB, 128))

        stats_rdma = rdma(st_send, st_recv, S_STATS, y_peer)
        stats_rdma.start()
        stats_rdma.wait()

        m_o = st_recv[:, 0:1]
        s_o = st_recv[:, 128:129]
        m_g = jnp.maximum(m, m_o)
        s_g = s * jnp.exp(m - m_g) + s_o * jnp.exp(m_o - m_g)
        e_ref[:, :] = e_ref[:, :] * (jnp.exp(m - m_g) / s_g)

        local = pltpu.make_async_copy(
            e_ref, out_ref.at[pl.ds(row0, RB), pl.ds(col0, CB)], copy_sem)
        local.start()
        z_a = rdma(e_ref, out_ref.at[pl.ds(row0, RB), pl.ds(col0, CB)],
                   S_Z, z_peer)
        y_a = rdma(e_ref, out_ref.at[pl.ds(row0, RB), pl.ds(col0, CB)],
                   S_YA, y_peer)
        x_a = rdma(e_ref, out_ref.at[pl.ds(row0, RB), pl.ds(col0, CB)],
                   S_XA, x_peer)
        z_a.start()
        y_a.start()
        x_a.start()

        z_a.wait_recv()
        y_b = rdma(out_ref.at[pl.ds(rowz, RB), pl.ds(col0, CB)],
                   out_ref.at[pl.ds(rowz, RB), pl.ds(col0, CB)],
                   S_YB, y_peer)
        x_b = rdma(out_ref.at[pl.ds(rowz, RB), pl.ds(col0, CB)],
                   out_ref.at[pl.ds(rowz, RB), pl.ds(col0, CB)],
                   S_XB, x_peer)
        y_b.start()
        x_b.start()

        y_a.wait_recv()
        x_ca = rdma(out_ref.at[pl.ds(row0, RB), pl.ds(colo, CB)],
                    out_ref.at[pl.ds(row0, RB), pl.ds(colo, CB)],
                    S_XCA, x_peer)
        x_ca.start()

        y_b.wait_recv()
        x_cb = rdma(out_ref.at[pl.ds(rowz, RB), pl.ds(colo, CB)],
                    out_ref.at[pl.ds(rowz, RB), pl.ds(colo, CB)],
                    S_XCB, x_peer)
        x_cb.start()

        x_a.wait_recv()
        x_b.wait_recv()
        x_ca.wait_recv()
        x_cb.wait_recv()
        local.wait()
        z_a.wait_send()
        y_a.wait_send()
        y_b.wait_send()
        x_a.wait_send()
        x_b.wait_send()
        x_ca.wait_send()
        x_cb.wait_send()

    return pl.pallas_call(
        body,
        out_shape=jax.ShapeDtypeStruct((T, V), jnp.float32),
        in_specs=[pl.BlockSpec(memory_space=pltpu.MemorySpace.VMEM)],
        out_specs=pl.BlockSpec(memory_space=pl.ANY),
        scratch_shapes=[
            pltpu.VMEM((RB, CB), jnp.float32),
            pltpu.VMEM((RB, 256), jnp.float32),
            pltpu.VMEM((RB, 256), jnp.float32),
            pltpu.SemaphoreType.DMA((8,)),
            pltpu.SemaphoreType.DMA((8,)),
            pltpu.SemaphoreType.DMA,
        ],
        compiler_params=pltpu.CompilerParams(collective_id=0),
    )(logits)


# device time: 894000 ns/iter; 1.6346x vs baseline; 1.0229x over previous
import jax
import jax.numpy as jnp
from jax import lax
from jax.experimental import pallas as pl
from jax.experimental.pallas import tpu as pltpu

T, D, V = 2048, 4096, 16384
RB = T // 4
CB = V // 2

S_STATS, S_Z, S_YA, S_YB, S_XA, S_XB, S_XCA, S_XCB = range(8)


def kernel(x, W):
    xi = lax.axis_index("x")
    yi = lax.axis_index("y")
    zi = lax.axis_index("z")
    r = 2 * xi + zi
    x_rows = lax.dynamic_slice_in_dim(x, r * RB, RB, axis=0)
    logits = jnp.dot(x_rows, W, preferred_element_type=jnp.float32)

    def body(logits_ref, out_ref, l_ref, e_ref, st_send, st_recv,
             send_sems, recv_sems, copy_sem, in_sem):
        xi = lax.axis_index("x")
        yi = lax.axis_index("y")
        zi = lax.axis_index("z")
        row0 = (2 * xi + zi) * RB
        rowz = (2 * xi + (1 - zi)) * RB
        col0 = yi * CB
        colo = (1 - yi) * CB
        z_peer = (xi, yi, 1 - zi)
        y_peer = (xi, 1 - yi, zi)
        x_peer = (1 - xi, yi, zi)

        def rdma(src, dst, sem_idx, peer):
            return pltpu.make_async_remote_copy(
                src_ref=src, dst_ref=dst,
                send_sem=send_sems.at[sem_idx], recv_sem=recv_sems.at[sem_idx],
                device_id=peer, device_id_type=pl.DeviceIdType.MESH)

        load = pltpu.make_async_copy(logits_ref, l_ref, in_sem)
        load.start()

        barrier = pltpu.get_barrier_semaphore()
        for nbr in [x_peer, y_peer, z_peer]:
            pl.semaphore_signal(barrier, inc=1, device_id=nbr,
                                device_id_type=pl.DeviceIdType.MESH)
        pl.semaphore_wait(barrier, 3)
        load.wait()

        l = l_ref[:, :]
        m = jnp.max(l, axis=1, keepdims=True)
        e = jnp.exp(l - m)
        e_ref[:, :] = e
        s = jnp.sum(e, axis=1, keepdims=True)
        st_send[:, 0:128] = jnp.broadcast_to(m, (RB, 128))
        st_send[:, 128:256] = jnp.broadcast_to(s, (RB, 128))

        stats_rdma = rdma(st_send, st_recv, S_STATS, y_peer)
        stats_rdma.start()
        stats_rdma.wait()

        m_o = st_recv[:, 0:1]
        s_o = st_recv[:, 128:129]
        m_g = jnp.maximum(m, m_o)
        s_g = s * jnp.exp(m - m_g) + s_o * jnp.exp(m_o - m_g)
        e_ref[:, :] = e_ref[:, :] * (jnp.exp(m - m_g) / s_g)

        local = pltpu.make_async_copy(
            e_ref, out_ref.at[pl.ds(row0, RB), pl.ds(col0, CB)], copy_sem)
        local.start()
        z_a = rdma(e_ref, out_ref.at[pl.ds(row0, RB), pl.ds(col0, CB)],
                   S_Z, z_peer)
        y_a = rdma(e_ref, out_ref.at[pl.ds(row0, RB), pl.ds(col0, CB)],
                   S_YA, y_peer)
        x_a = rdma(e_ref, out_ref.at[pl.ds(row0, RB), pl.ds(col0, CB)],
                   S_XA, x_peer)
        z_a.start()
        y_a.start()
        x_a.start()

        z_a.wait_recv()
        y_b = rdma(out_ref.at[pl.ds(rowz, RB), pl.ds(col0, CB)],
                   out_ref.at[pl.ds(rowz, RB), pl.ds(col0, CB)],
                   S_YB, y_peer)
        x_b = rdma(out_ref.at[pl.ds(rowz, RB), pl.ds(col0, CB)],
                   out_ref.at[pl.ds(rowz, RB), pl.ds(col0, CB)],
                   S_XB, x_peer)
        y_b.start()
        x_b.start()

        y_a.wait_recv()
        x_ca = rdma(out_ref.at[pl.ds(row0, RB), pl.ds(colo, CB)],
                    out_ref.at[pl.ds(row0, RB), pl.ds(colo, CB)],
                    S_XCA, x_peer)
        x_ca.start()

        y_b.wait_recv()
        x_cb = rdma(out_ref.at[pl.ds(rowz, RB), pl.ds(colo, CB)],
                    out_ref.at[pl.ds(rowz, RB), pl.ds(colo, CB)],
                    S_XCB, x_peer)
        x_cb.start()

        x_a.wait_recv()
        x_b.wait_recv()
        x_ca.wait_recv()
        x_cb.wait_recv()
        local.wait()
        z_a.wait_send()
        y_a.wait_send()
        y_b.wait_send()
        x_a.wait_send()
        x_b.wait_send()
        x_ca.wait_send()
        x_cb.wait_send()

    return pl.pallas_call(
        body,
        out_shape=jax.ShapeDtypeStruct((T, V), jnp.float32),
        in_specs=[pl.BlockSpec(memory_space=pl.ANY)],
        out_specs=pl.BlockSpec(memory_space=pl.ANY),
        scratch_shapes=[
            pltpu.VMEM((RB, CB), jnp.float32),
            pltpu.VMEM((RB, CB), jnp.float32),
            pltpu.VMEM((RB, 256), jnp.float32),
            pltpu.VMEM((RB, 256), jnp.float32),
            pltpu.SemaphoreType.DMA((8,)),
            pltpu.SemaphoreType.DMA((8,)),
            pltpu.SemaphoreType.DMA,
            pltpu.SemaphoreType.DMA,
        ],
        compiler_params=pltpu.CompilerParams(
            collective_id=0, vmem_limit_bytes=60 * 1024 * 1024),
    )(logits)
